# baseline (speedup 1.0000x reference)
import jax
import jax.numpy as jnp
from jax.experimental import pallas as pl
from jax.experimental.pallas import tpu as pltpu

_B, _H, _W, _J = 32, 512, 512, 17
_IMGS = 4
_CH = 8


def _body(xs_ref, ys_ref, m_ref, g_ref, out_ref, acc_ref):
    s = pl.program_id(0)

    @pl.when(s == 0)
    def _init():
        out_ref[3] = 0.0
        acc_ref[...] = jnp.zeros_like(acc_ref)

    zero_c = jnp.zeros((_CH, _W), jnp.float32)

    def chunk(c, carry):
        a_pos, a_cnt, a_all = carry
        x = m_ref[pl.ds(c * _CH, _CH), :]
        g = g_ref[pl.ds(c * _CH, _CH), :]
        l_term = jnp.log1p(jnp.exp(-jnp.abs(x)))
        pos_val = jnp.maximum(-x, 0.0) + l_term
        pos = g > 0.0
        a_pos = a_pos + jnp.where(pos, pos_val, zero_c)
        a_cnt = a_cnt + jnp.where(pos, 1.0, 0.0)
        a_all = a_all + (pos_val + x)
        return a_pos, a_cnt, a_all

    a_pos, a_cnt, a_all = jax.lax.fori_loop(
        0, (_IMGS * _H) // _CH, chunk,
        (acc_ref[0], acc_ref[1], acc_ref[2]), unroll=4)
    acc_ref[0] = a_pos
    acc_ref[1] = a_cnt
    acc_ref[2] = a_all

    cols = jax.lax.broadcasted_iota(jnp.int32, (_CH, _W), 1)
    a_ins = acc_ref[3]
    for i in range(_IMGS):
        b = s * _IMGS + i
        xrow = xs_ref[pl.ds(b, 1), :]
        yrow = ys_ref[pl.ds(b, 1), :]
        x_min = jnp.maximum(jnp.min(xrow).astype(jnp.int32) - 10, 0)
        x_max = jnp.minimum(jnp.max(xrow).astype(jnp.int32) + 10, _W)
        y_min = jnp.maximum(jnp.min(yrow).astype(jnp.int32) - 10, 0)
        y_max = jnp.minimum(jnp.max(yrow).astype(jnp.int32) + 10, _H)
        y_len = jnp.maximum(y_max - y_min, 0)
        x_len = jnp.maximum(x_max - x_min, 0)

        col_in = (cols - x_min).astype(jnp.uint32) < x_len.astype(jnp.uint32)
        row0 = i * _H
        base = row0 + y_min
        lo = row0 // _CH + y_min // _CH
        hi = jnp.where(y_len > 0, row0 // _CH + (y_max + _CH - 1) // _CH, lo)

        def ins_chunk(j, a, base=base, y_len=y_len, col_in=col_in):
            xx = m_ref[pl.ds(j * _CH, _CH), :]
            l_term = jnp.log1p(jnp.exp(-jnp.abs(xx)))
            neg_val = jnp.maximum(xx, 0.0) + l_term
            rows = j * _CH + jax.lax.broadcasted_iota(
                jnp.int32, (_CH, _W), 0)
            row_in = (rows - base).astype(jnp.uint32) < y_len.astype(
                jnp.uint32)
            return a + jnp.where(row_in & col_in, neg_val, zero_c)

        a_ins = jax.lax.fori_loop(lo, hi, ins_chunk, a_ins)
        out_ref[3] += (y_len * x_len).astype(jnp.float32)
    acc_ref[3] = a_ins

    @pl.when(s == pl.num_programs(0) - 1)
    def _finish():
        out_ref[0] = jnp.sum(acc_ref[0])
        out_ref[1] = jnp.sum(acc_ref[1])
        out_ref[2] = jnp.sum(acc_ref[2]) - jnp.sum(acc_ref[3])


def kernel(skls, masks, gt_masks):
    s = jax.lax.stop_gradient(skls)
    xs = s[:, :, 0]
    ys = s[:, :, 1]
    m2d = masks.reshape(_B * _H, _W)
    g2d = gt_masks.reshape(_B * _H, _W)

    acc = pl.pallas_call(
        _body,
        grid=(_B // _IMGS,),
        in_specs=[
            pl.BlockSpec((_B, _J), lambda s: (0, 0)),
            pl.BlockSpec((_B, _J), lambda s: (0, 0)),
            pl.BlockSpec((_IMGS * _H, _W), lambda s: (s, 0)),
            pl.BlockSpec((_IMGS * _H, _W), lambda s: (s, 0)),
        ],
        out_specs=pl.BlockSpec(memory_space=pltpu.SMEM),
        out_shape=jax.ShapeDtypeStruct((4,), jnp.float32),
        scratch_shapes=[pltpu.VMEM((4, _CH, _W), jnp.float32)],
        compiler_params=pltpu.CompilerParams(
            dimension_semantics=("arbitrary",),
        ),
    )(xs, ys, m2d, g2d)

    pos_loss = acc[0] / jnp.maximum(acc[1], 1.0)
    neg_count = float(_B * _H * _W) - acc[3]
    neg_loss = acc[2] / jnp.maximum(neg_count, 1.0)
    return pos_loss + neg_loss
